# baseline (speedup 1.0000x reference)
import jax
import jax.numpy as jnp
from jax.experimental import pallas as pl


def _edge_mlp_body(src_ref, dst_ref, ea_ref, w1a_ref, w1b_ref, w1c_ref,
                   b1_ref, w2_ref, b2_ref, out_ref):
    h = (src_ref[...] @ w1a_ref[...]
         + dst_ref[...] @ w1b_ref[...]
         + ea_ref[...] @ w1c_ref[...]
         + b1_ref[...])
    h = jnp.maximum(h, 0.0)
    out_ref[...] = h @ w2_ref[...] + b2_ref[...]


def _node_mlp_body(x_ref, agg_ref, wn1a_ref, wn1b_ref, bn1_ref, wn2_ref,
                   bn2_ref, wga_ref, wgb_ref, bg_ref, gate_ref, xnew_ref):
    x = x_ref[...]
    agg = agg_ref[...]
    hn = jnp.maximum(x @ wn1a_ref[...] + agg @ wn1b_ref[...] + bn1_ref[...], 0.0)
    x_upd = hn @ wn2_ref[...] + bn2_ref[...]
    gate = jax.nn.sigmoid(x @ wga_ref[...] + agg @ wgb_ref[...] + bg_ref[...])
    gate_ref[...] = gate
    xnew_ref[...] = gate * x_upd


def kernel(x, edge_index, edge_attr, u, batch, W_e1, b_e1, W_e2, b_e2,
           W_n1, b_n1, W_n2, b_n2, W_g, b_g, W_u1, b_u1, W_u2, b_u2):
    N, D = x.shape
    E = edge_index.shape[1]
    H = W_e2.shape[1]
    B, G = u.shape

    src = jnp.take(x, edge_index[0], axis=0)
    dst = jnp.take(x, edge_index[1], axis=0)

    W1a, W1b, W1c = W_e1[:D], W_e1[D:2 * D], W_e1[2 * D:]
    BE = 2000
    grid_e = E // BE
    b1 = b_e1.reshape(1, H)
    b2 = b_e2.reshape(1, H)
    message = pl.pallas_call(
        _edge_mlp_body,
        grid=(grid_e,),
        in_specs=[
            pl.BlockSpec((BE, D), lambda i: (i, 0)),
            pl.BlockSpec((BE, D), lambda i: (i, 0)),
            pl.BlockSpec((BE, edge_attr.shape[1]), lambda i: (i, 0)),
            pl.BlockSpec((D, H), lambda i: (0, 0)),
            pl.BlockSpec((D, H), lambda i: (0, 0)),
            pl.BlockSpec((edge_attr.shape[1], H), lambda i: (0, 0)),
            pl.BlockSpec((1, H), lambda i: (0, 0)),
            pl.BlockSpec((H, H), lambda i: (0, 0)),
            pl.BlockSpec((1, H), lambda i: (0, 0)),
        ],
        out_specs=pl.BlockSpec((BE, H), lambda i: (i, 0)),
        out_shape=jax.ShapeDtypeStruct((E, H), jnp.float32),
    )(src, dst, edge_attr, W1a, W1b, W1c, b1, W_e2, b2)

    agg = jax.ops.segment_sum(message, edge_index[1], num_segments=N)

    Wn1a, Wn1b = W_n1[:D], W_n1[D:]
    Wga, Wgb = W_g[:D], W_g[D:]
    BN = 1000
    grid_n = N // BN
    gate, x_new = pl.pallas_call(
        _node_mlp_body,
        grid=(grid_n,),
        in_specs=[
            pl.BlockSpec((BN, D), lambda i: (i, 0)),
            pl.BlockSpec((BN, H), lambda i: (i, 0)),
            pl.BlockSpec((D, H), lambda i: (0, 0)),
            pl.BlockSpec((H, H), lambda i: (0, 0)),
            pl.BlockSpec((1, H), lambda i: (0, 0)),
            pl.BlockSpec((H, H), lambda i: (0, 0)),
            pl.BlockSpec((1, H), lambda i: (0, 0)),
            pl.BlockSpec((D, H), lambda i: (0, 0)),
            pl.BlockSpec((H, H), lambda i: (0, 0)),
            pl.BlockSpec((1, H), lambda i: (0, 0)),
        ],
        out_specs=[
            pl.BlockSpec((BN, H), lambda i: (i, 0)),
            pl.BlockSpec((BN, H), lambda i: (i, 0)),
        ],
        out_shape=[
            jax.ShapeDtypeStruct((N, H), jnp.float32),
            jax.ShapeDtypeStruct((N, H), jnp.float32),
        ],
    )(x, agg, Wn1a, Wn1b, b_n1.reshape(1, H), W_n2, b_n2.reshape(1, H),
      Wga, Wgb, b_g.reshape(1, H))

    pooled = jax.ops.segment_sum(x_new, batch, num_segments=B)
    counts = jax.ops.segment_sum(jnp.ones((N,), jnp.float32), batch,
                                 num_segments=B)
    pooled = pooled / jnp.maximum(counts, 1.0)[:, None]
    g_in = jnp.concatenate([pooled, u], axis=-1)
    hg = jax.nn.relu(g_in @ W_u1 + b_u1)
    u_new = hg @ W_u2 + b_u2
    return (u_new, gate)
